# Initial kernel scaffold; baseline (speedup 1.0000x reference)
import jax
import jax.numpy as jnp
from jax.experimental import pallas as pl


def kernel(lf, mf, sf, da_preds, da_images, da_labels, device, lf_W1, lf_b1, lf_W2, lf_b2, mf_W1, mf_b1, mf_W2, mf_b2, sf_W1, sf_b1, sf_W2, sf_b2):
    raise NotImplementedError("write your pallas kernel here")


# speedup vs baseline: 1.4889x; 1.4889x over previous
import numpy as np
import jax
import jax.numpy as jnp
from jax.experimental import pallas as pl

_ITERS = 10
_B = 8
_OC = 32


def _kmeans_body(x_ref, out_ref):
    x = x_ref[0]
    n = x.shape[0]
    c0 = x[0:1, :]
    c1 = x[1:2, :]
    total = jnp.sum(x, axis=0, keepdims=True)
    cnt0 = jnp.float32(n)
    cnt1 = jnp.float32(0.0)
    for _ in range(_ITERS):
        d0 = jnp.sum((x - c0) ** 2, axis=1, keepdims=True)
        d1 = jnp.sum((x - c1) ** 2, axis=1, keepdims=True)
        m1 = (d1 < d0).astype(jnp.float32)
        cnt1 = jnp.sum(m1)
        cnt0 = n - cnt1
        s1 = jnp.sum(x * m1, axis=0, keepdims=True)
        s0 = total - s1
        c0 = s0 / jnp.maximum(cnt0, 1.0)
        c1 = s1 / jnp.maximum(cnt1, 1.0)
    out_ref[0] = jnp.where(cnt1 > cnt0, c1, c0)


def _kmeans_center(x):
    b, n, hw = x.shape
    return pl.pallas_call(
        _kmeans_body,
        grid=(b,),
        in_specs=[pl.BlockSpec((1, n, hw), lambda i: (i, 0, 0))],
        out_specs=pl.BlockSpec((1, 1, hw), lambda i: (i, 0, 0)),
        out_shape=jax.ShapeDtypeStruct((b, 1, hw), jnp.float32),
    )(x)


def _sel_mats(h):
    h1 = h // 2
    mats = []
    for d in range(4):
        s = np.zeros((h1, h), np.float32)
        for i in range(h1):
            j = 2 * i + d - 1
            if 0 <= j < h:
                s[i, j] = 1.0
        mats.append(s)
    return mats


def _comb_vecs(h1):
    h2 = h1 // 2
    vecs = []
    for d in range(4):
        u = np.zeros((1, h1), np.float32)
        for o in range(h2):
            j = 2 * o + d - 1
            if 0 <= j < h1:
                u[0, j] = 1.0
        vecs.append(u)
    return vecs


def _make_disc_body(h):
    h1 = h // 2
    h2 = h // 4

    def body(x_ref, w1_ref, b1_ref, w2_ref, b2_ref, lab_ref, s_ref, u_ref,
             out_ref):
        i = pl.program_id(0)
        x = x_ref[0]
        w1 = w1_ref[...]
        b1 = b1_ref[...]
        w2 = w2_ref[...]
        S = [s_ref[d * h1 : (d + 1) * h1, :] for d in range(4)]
        U = [u_ref[d : d + 1, :] for d in range(4)]

        y1 = jnp.zeros((_OC, h1, h1), jnp.float32)
        for dy in range(4):
            a = jnp.dot(S[dy], x)
            for dx in range(4):
                p = jnp.dot(a, S[dx].T)
                t = 4 * dy + dx
                y1 = y1 + p[None] * w1[:, t : t + 1][:, :, None]
        y1 = y1 + b1[:, :, None]
        y1 = jnp.where(y1 >= 0, y1, 0.2 * y1)

        acc = jnp.float32(0.0)
        for dy in range(4):
            r = jnp.sum(y1 * U[dy][:, :, None], axis=1)
            for dx in range(4):
                tv = jnp.sum(r * U[dx], axis=1, keepdims=True)
                acc = acc + jnp.sum(tv * w2[:, 4 * dy + dx : 4 * dy + dx + 1])
        logit = acc / jnp.float32(h2 * h2) + b2_ref[0, 0]

        tgt = lab_ref[0, 0, 0]
        g = (
            jnp.maximum(logit, 0.0)
            - logit * tgt
            + jnp.log1p(jnp.exp(-jnp.abs(logit)))
        )
        contrib = g / jnp.float32(_B)

        contrib2d = jnp.broadcast_to(contrib, (1, 1))

        @pl.when(i == 0)
        def _():
            out_ref[...] = contrib2d

        @pl.when(i > 0)
        def _():
            out_ref[...] = out_ref[...] + contrib2d

    return body


def _disc_loss(maps, w1, b1, w2, b2, labels):
    b, h, _ = maps.shape
    h1 = h // 2
    s_all = jnp.asarray(np.concatenate(_sel_mats(h), axis=0))
    u_all = jnp.asarray(np.concatenate(_comb_vecs(h1), axis=0))
    out = pl.pallas_call(
        _make_disc_body(h),
        grid=(b,),
        in_specs=[
            pl.BlockSpec((1, h, h), lambda i: (i, 0, 0)),
            pl.BlockSpec((_OC, 16), lambda i: (0, 0)),
            pl.BlockSpec((_OC, 1), lambda i: (0, 0)),
            pl.BlockSpec((_OC, 16), lambda i: (0, 0)),
            pl.BlockSpec((1, 1), lambda i: (0, 0)),
            pl.BlockSpec((1, 1, 1), lambda i: (i, 0, 0)),
            pl.BlockSpec((4 * h1, h), lambda i: (0, 0)),
            pl.BlockSpec((4, h1), lambda i: (0, 0)),
        ],
        out_specs=pl.BlockSpec((1, 1), lambda i: (0, 0)),
        out_shape=jax.ShapeDtypeStruct((1, 1), jnp.float32),
    )(maps, w1, b1, w2, b2, labels, s_all, u_all)
    return out[0, 0]


def kernel(lf, mf, sf, da_preds, da_images, da_labels, device,
           lf_W1, lf_b1, lf_W2, lf_b2,
           mf_W1, mf_b1, mf_W2, mf_b2,
           sf_W1, sf_b1, sf_W2, sf_b2):
    labels = da_labels.astype(jnp.float32).reshape(_B, 1, 1)
    outs = []
    for feats, W1, b1, W2, b2 in (
        (lf, lf_W1, lf_b1, lf_W2, lf_b2),
        (mf, mf_W1, mf_b1, mf_W2, mf_b2),
        (sf, sf_W1, sf_b1, sf_W2, sf_b2),
    ):
        b, c, h, w = feats.shape
        x = feats.reshape(b, c, h * w)
        centers = _kmeans_center(x)
        maps = centers.reshape(b, h, w)
        loss = _disc_loss(
            maps,
            W1.reshape(_OC, 16),
            b1.reshape(_OC, 1),
            W2.reshape(_OC, 16),
            b2.reshape(1, 1),
            labels,
        )
        outs.append(loss)
    return tuple(outs)


# speedup vs baseline: 1.8070x; 1.2136x over previous
import numpy as np
import jax
import jax.numpy as jnp
from jax.experimental import pallas as pl

_ITERS = 10
_B = 8
_OC = 32
_HS = (32, 16, 8)


def _kmeans_body(x_ref, out_ref):
    x = x_ref[0]
    n = x.shape[0]
    c0 = x[0:1, :]
    c1 = x[1:2, :]
    total = jnp.sum(x, axis=0, keepdims=True)
    cnt0 = jnp.float32(n)
    cnt1 = jnp.float32(0.0)
    for _ in range(_ITERS):
        w = c0 - c1
        thr = 0.5 * jnp.sum(w * (c0 + c1))
        e = jnp.sum(x * w, axis=1, keepdims=True)
        m1 = (e < thr).astype(jnp.float32)
        cnt1 = jnp.sum(m1)
        cnt0 = n - cnt1
        s1 = jnp.sum(x * m1, axis=0, keepdims=True)
        s0 = total - s1
        c0 = s0 / jnp.maximum(cnt0, 1.0)
        c1 = s1 / jnp.maximum(cnt1, 1.0)
    out_ref[0] = jnp.where(cnt1 > cnt0, c1, c0)


def _kmeans_center(x):
    b, n, hw = x.shape
    return pl.pallas_call(
        _kmeans_body,
        grid=(b,),
        in_specs=[pl.BlockSpec((1, n, hw), lambda i: (i, 0, 0))],
        out_specs=pl.BlockSpec((1, 1, hw), lambda i: (i, 0, 0)),
        out_shape=jax.ShapeDtypeStruct((b, 1, hw), jnp.float32),
    )(x)


def _sel_mats(h):
    h1 = h // 2
    mats = []
    for d in range(4):
        s = np.zeros((h1, h), np.float32)
        for i in range(h1):
            j = 2 * i + d - 1
            if 0 <= j < h:
                s[i, j] = 1.0
        mats.append(s)
    return mats


def _comb_vecs(h1):
    h2 = h1 // 2
    vecs = []
    for d in range(4):
        u = np.zeros((1, h1), np.float32)
        for o in range(h2):
            j = 2 * o + d - 1
            if 0 <= j < h1:
                u[0, j] = 1.0
        vecs.append(u)
    return vecs


def _disc_logit(cmap, w1, b1, w2, b2, s_ref, u_ref, h):
    h1 = h // 2
    h2 = h // 4
    S = [s_ref[d * h1 : (d + 1) * h1, :] for d in range(4)]
    U = [u_ref[d : d + 1, :] for d in range(4)]

    y1 = jnp.zeros((_OC, h1, h1), jnp.float32)
    for dy in range(4):
        a = jnp.dot(S[dy], cmap)
        for dx in range(4):
            p = jnp.dot(a, S[dx].T)
            t = 4 * dy + dx
            y1 = y1 + p[None] * w1[:, t : t + 1][:, :, None]
    y1 = y1 + b1[:, :, None]
    y1 = jnp.where(y1 >= 0, y1, 0.2 * y1)

    acc = jnp.float32(0.0)
    for dy in range(4):
        r = jnp.sum(y1 * U[dy][:, :, None], axis=1)
        for dx in range(4):
            tv = jnp.sum(r * U[dx], axis=1, keepdims=True)
            acc = acc + jnp.sum(tv * w2[:, 4 * dy + dx : 4 * dy + dx + 1])
    return acc / jnp.float32(h2 * h2) + b2[0, 0]


def _bce_term(logit, tgt):
    return (
        jnp.maximum(logit, 0.0)
        - logit * tgt
        + jnp.log1p(jnp.exp(-jnp.abs(logit)))
    ) / jnp.float32(_B)


def _disc_body(xl_ref, xm_ref, xs_ref,
               w1l_ref, b1l_ref, w2l_ref, b2l_ref,
               w1m_ref, b1m_ref, w2m_ref, b2m_ref,
               w1s_ref, b1s_ref, w2s_ref, b2s_ref,
               lab_ref,
               sl_ref, ul_ref, sm_ref, um_ref, ss_ref, us_ref,
               ol_ref, om_ref, os_ref):
    i = pl.program_id(0)
    tgt = lab_ref[0, 0, 0]
    triples = (
        (xl_ref, w1l_ref, b1l_ref, w2l_ref, b2l_ref, sl_ref, ul_ref, ol_ref),
        (xm_ref, w1m_ref, b1m_ref, w2m_ref, b2m_ref, sm_ref, um_ref, om_ref),
        (xs_ref, w1s_ref, b1s_ref, w2s_ref, b2s_ref, ss_ref, us_ref, os_ref),
    )
    for h, (x_ref, w1_ref, b1_ref, w2_ref, b2_ref, s_ref, u_ref, o_ref) in zip(
        _HS, triples
    ):
        logit = _disc_logit(
            x_ref[0], w1_ref[...], b1_ref[...], w2_ref[...], b2_ref[...],
            s_ref, u_ref, h,
        )
        contrib2d = jnp.broadcast_to(_bce_term(logit, tgt), (1, 1))

        @pl.when(i == 0)
        def _():
            o_ref[...] = contrib2d

        @pl.when(i > 0)
        def _():
            o_ref[...] = o_ref[...] + contrib2d


def _disc_losses(maps, weights, labels):
    b = maps[0].shape[0]
    sel = []
    for h in _HS:
        sel.append(jnp.asarray(np.concatenate(_sel_mats(h), axis=0)))
        sel.append(jnp.asarray(np.concatenate(_comb_vecs(h // 2), axis=0)))
    in_specs = [
        pl.BlockSpec((1, h, h), lambda i: (i, 0, 0)) for h in _HS
    ]
    wargs = []
    for w1, b1, w2, b2 in weights:
        wargs += [w1, b1, w2, b2]
        in_specs += [
            pl.BlockSpec((_OC, 16), lambda i: (0, 0)),
            pl.BlockSpec((_OC, 1), lambda i: (0, 0)),
            pl.BlockSpec((_OC, 16), lambda i: (0, 0)),
            pl.BlockSpec((1, 1), lambda i: (0, 0)),
        ]
    in_specs.append(pl.BlockSpec((1, 1, 1), lambda i: (i, 0, 0)))
    for h in _HS:
        in_specs += [
            pl.BlockSpec((2 * h, h), lambda i: (0, 0)),
            pl.BlockSpec((4, h // 2), lambda i: (0, 0)),
        ]
    outs = pl.pallas_call(
        _disc_body,
        grid=(b,),
        in_specs=in_specs,
        out_specs=[pl.BlockSpec((1, 1), lambda i: (0, 0))] * 3,
        out_shape=[jax.ShapeDtypeStruct((1, 1), jnp.float32)] * 3,
    )(*maps, *wargs, labels, *sel)
    return tuple(o[0, 0] for o in outs)


def kernel(lf, mf, sf, da_preds, da_images, da_labels, device,
           lf_W1, lf_b1, lf_W2, lf_b2,
           mf_W1, mf_b1, mf_W2, mf_b2,
           sf_W1, sf_b1, sf_W2, sf_b2):
    labels = da_labels.astype(jnp.float32).reshape(_B, 1, 1)
    maps = []
    for feats in (lf, mf, sf):
        b, c, h, w = feats.shape
        x = feats.reshape(b, c, h * w)
        centers = _kmeans_center(x)
        maps.append(centers.reshape(b, h, w))
    weights = [
        (lf_W1.reshape(_OC, 16), lf_b1.reshape(_OC, 1),
         lf_W2.reshape(_OC, 16), lf_b2.reshape(1, 1)),
        (mf_W1.reshape(_OC, 16), mf_b1.reshape(_OC, 1),
         mf_W2.reshape(_OC, 16), mf_b2.reshape(1, 1)),
        (sf_W1.reshape(_OC, 16), sf_b1.reshape(_OC, 1),
         sf_W2.reshape(_OC, 16), sf_b2.reshape(1, 1)),
    ]
    return _disc_losses(maps, weights, labels)


# speedup vs baseline: 1.8571x; 1.0277x over previous
import numpy as np
import jax
import jax.numpy as jnp
from jax.experimental import pallas as pl

_ITERS = 10
_B = 8
_OC = 32
_HS = (32, 16, 8)


def _kmeans_one(x):
    n = x.shape[0]
    c0 = x[0:1, :]
    c1 = x[1:2, :]
    total = jnp.sum(x, axis=0, keepdims=True)
    cnt0 = jnp.float32(n)
    cnt1 = jnp.float32(0.0)
    for _ in range(_ITERS):
        w = c0 - c1
        thr = 0.5 * jnp.sum(w * (c0 + c1))
        e = jnp.sum(x * w, axis=1, keepdims=True)
        m1 = (e < thr).astype(jnp.float32)
        cnt1 = jnp.sum(m1)
        cnt0 = n - cnt1
        s1 = jnp.sum(x * m1, axis=0, keepdims=True)
        s0 = total - s1
        c0 = s0 / jnp.maximum(cnt0, 1.0)
        c1 = s1 / jnp.maximum(cnt1, 1.0)
    return jnp.where(cnt1 > cnt0, c1, c0)


def _kmeans_body(xl_ref, xm_ref, xs_ref, ol_ref, om_ref, os_ref):
    ol_ref[0] = _kmeans_one(xl_ref[0])
    om_ref[0] = _kmeans_one(xm_ref[0])
    os_ref[0] = _kmeans_one(xs_ref[0])


def _kmeans_centers(xl, xm, xs):
    b = xl.shape[0]
    return pl.pallas_call(
        _kmeans_body,
        grid=(b,),
        in_specs=[
            pl.BlockSpec((1, x.shape[1], x.shape[2]), lambda i: (i, 0, 0))
            for x in (xl, xm, xs)
        ],
        out_specs=[
            pl.BlockSpec((1, 1, x.shape[2]), lambda i: (i, 0, 0))
            for x in (xl, xm, xs)
        ],
        out_shape=[
            jax.ShapeDtypeStruct((b, 1, x.shape[2]), jnp.float32)
            for x in (xl, xm, xs)
        ],
    )(xl, xm, xs)


def _sel_mats(h):
    h1 = h // 2
    mats = []
    for d in range(4):
        s = np.zeros((h1, h), np.float32)
        for i in range(h1):
            j = 2 * i + d - 1
            if 0 <= j < h:
                s[i, j] = 1.0
        mats.append(s)
    return mats


def _comb_vecs(h1):
    h2 = h1 // 2
    vecs = []
    for d in range(4):
        u = np.zeros((1, h1), np.float32)
        for o in range(h2):
            j = 2 * o + d - 1
            if 0 <= j < h1:
                u[0, j] = 1.0
        vecs.append(u)
    return vecs


def _disc_logit(cmap, w1, b1, w2, b2, s_ref, u_ref, h):
    h1 = h // 2
    h2 = h // 4
    S = [s_ref[d * h1 : (d + 1) * h1, :] for d in range(4)]
    U = [u_ref[d : d + 1, :] for d in range(4)]

    y1 = jnp.zeros((_OC, h1, h1), jnp.float32)
    for dy in range(4):
        a = jnp.dot(S[dy], cmap)
        for dx in range(4):
            p = jnp.dot(a, S[dx].T)
            t = 4 * dy + dx
            y1 = y1 + p[None] * w1[:, t : t + 1][:, :, None]
    y1 = y1 + b1[:, :, None]
    y1 = jnp.where(y1 >= 0, y1, 0.2 * y1)

    acc = jnp.float32(0.0)
    for dy in range(4):
        r = jnp.sum(y1 * U[dy][:, :, None], axis=1)
        for dx in range(4):
            tv = jnp.sum(r * U[dx], axis=1, keepdims=True)
            acc = acc + jnp.sum(tv * w2[:, 4 * dy + dx : 4 * dy + dx + 1])
    return acc / jnp.float32(h2 * h2) + b2[0, 0]


def _bce_term(logit, tgt):
    return (
        jnp.maximum(logit, 0.0)
        - logit * tgt
        + jnp.log1p(jnp.exp(-jnp.abs(logit)))
    ) / jnp.float32(_B)


def _disc_body(xl_ref, xm_ref, xs_ref,
               w1l_ref, b1l_ref, w2l_ref, b2l_ref,
               w1m_ref, b1m_ref, w2m_ref, b2m_ref,
               w1s_ref, b1s_ref, w2s_ref, b2s_ref,
               lab_ref,
               sl_ref, ul_ref, sm_ref, um_ref, ss_ref, us_ref,
               ol_ref, om_ref, os_ref):
    i = pl.program_id(0)
    tgt = lab_ref[0, 0, 0]
    triples = (
        (xl_ref, w1l_ref, b1l_ref, w2l_ref, b2l_ref, sl_ref, ul_ref, ol_ref),
        (xm_ref, w1m_ref, b1m_ref, w2m_ref, b2m_ref, sm_ref, um_ref, om_ref),
        (xs_ref, w1s_ref, b1s_ref, w2s_ref, b2s_ref, ss_ref, us_ref, os_ref),
    )
    for h, (x_ref, w1_ref, b1_ref, w2_ref, b2_ref, s_ref, u_ref, o_ref) in zip(
        _HS, triples
    ):
        logit = _disc_logit(
            x_ref[0], w1_ref[...], b1_ref[...], w2_ref[...], b2_ref[...],
            s_ref, u_ref, h,
        )
        contrib2d = jnp.broadcast_to(_bce_term(logit, tgt), (1, 1))

        @pl.when(i == 0)
        def _():
            o_ref[...] = contrib2d

        @pl.when(i > 0)
        def _():
            o_ref[...] = o_ref[...] + contrib2d


def _disc_losses(maps, weights, labels):
    b = maps[0].shape[0]
    sel = []
    for h in _HS:
        sel.append(jnp.asarray(np.concatenate(_sel_mats(h), axis=0)))
        sel.append(jnp.asarray(np.concatenate(_comb_vecs(h // 2), axis=0)))
    in_specs = [
        pl.BlockSpec((1, h, h), lambda i: (i, 0, 0)) for h in _HS
    ]
    wargs = []
    for w1, b1, w2, b2 in weights:
        wargs += [w1, b1, w2, b2]
        in_specs += [
            pl.BlockSpec((_OC, 16), lambda i: (0, 0)),
            pl.BlockSpec((_OC, 1), lambda i: (0, 0)),
            pl.BlockSpec((_OC, 16), lambda i: (0, 0)),
            pl.BlockSpec((1, 1), lambda i: (0, 0)),
        ]
    in_specs.append(pl.BlockSpec((1, 1, 1), lambda i: (i, 0, 0)))
    for h in _HS:
        in_specs += [
            pl.BlockSpec((2 * h, h), lambda i: (0, 0)),
            pl.BlockSpec((4, h // 2), lambda i: (0, 0)),
        ]
    outs = pl.pallas_call(
        _disc_body,
        grid=(b,),
        in_specs=in_specs,
        out_specs=[pl.BlockSpec((1, 1), lambda i: (0, 0))] * 3,
        out_shape=[jax.ShapeDtypeStruct((1, 1), jnp.float32)] * 3,
    )(*maps, *wargs, labels, *sel)
    return tuple(o[0, 0] for o in outs)


def kernel(lf, mf, sf, da_preds, da_images, da_labels, device,
           lf_W1, lf_b1, lf_W2, lf_b2,
           mf_W1, mf_b1, mf_W2, mf_b2,
           sf_W1, sf_b1, sf_W2, sf_b2):
    labels = da_labels.astype(jnp.float32).reshape(_B, 1, 1)
    xs = [f.reshape(f.shape[0], f.shape[1], -1) for f in (lf, mf, sf)]
    centers = _kmeans_centers(*xs)
    maps = [
        c.reshape(f.shape[0], f.shape[2], f.shape[3])
        for c, f in zip(centers, (lf, mf, sf))
    ]
    weights = [
        (lf_W1.reshape(_OC, 16), lf_b1.reshape(_OC, 1),
         lf_W2.reshape(_OC, 16), lf_b2.reshape(1, 1)),
        (mf_W1.reshape(_OC, 16), mf_b1.reshape(_OC, 1),
         mf_W2.reshape(_OC, 16), mf_b2.reshape(1, 1)),
        (sf_W1.reshape(_OC, 16), sf_b1.reshape(_OC, 1),
         sf_W2.reshape(_OC, 16), sf_b2.reshape(1, 1)),
    ]
    return _disc_losses(maps, weights, labels)


# speedup vs baseline: 1.8579x; 1.0004x over previous
import numpy as np
import jax
import jax.numpy as jnp
from jax.experimental import pallas as pl

_ITERS = 10
_B = 8
_OC = 32
_HS = (32, 16, 8)


def _kmeans_one(x):
    n = x.shape[0]
    c0 = x[0:1, :]
    c1 = x[1:2, :]
    total = jnp.sum(x, axis=0, keepdims=True)
    cnt0 = jnp.float32(n)
    cnt1 = jnp.float32(0.0)
    for _ in range(_ITERS):
        w = c0 - c1
        thr = 0.5 * jnp.sum(w * (c0 + c1))
        e = jnp.sum(x * w, axis=1, keepdims=True)
        m1 = (e < thr).astype(jnp.float32)
        cnt1 = jnp.sum(m1)
        cnt0 = n - cnt1
        s1 = jnp.sum(x * m1, axis=0, keepdims=True)
        s0 = total - s1
        c0 = s0 / jnp.maximum(cnt0, 1.0)
        c1 = s1 / jnp.maximum(cnt1, 1.0)
    return jnp.where(cnt1 > cnt0, c1, c0)


_SPB = 1


def _kmeans_body(xl_ref, xm_ref, xs_ref, ol_ref, om_ref, os_ref):
    for s in range(_SPB):
        ol_ref[s] = _kmeans_one(xl_ref[s])
        om_ref[s] = _kmeans_one(xm_ref[s])
        os_ref[s] = _kmeans_one(xs_ref[s])


def _kmeans_centers(xl, xm, xs):
    b = xl.shape[0]
    return pl.pallas_call(
        _kmeans_body,
        grid=(b // _SPB,),
        in_specs=[
            pl.BlockSpec((_SPB, x.shape[1], x.shape[2]), lambda i: (i, 0, 0))
            for x in (xl, xm, xs)
        ],
        out_specs=[
            pl.BlockSpec((_SPB, 1, x.shape[2]), lambda i: (i, 0, 0))
            for x in (xl, xm, xs)
        ],
        out_shape=[
            jax.ShapeDtypeStruct((b, 1, x.shape[2]), jnp.float32)
            for x in (xl, xm, xs)
        ],
    )(xl, xm, xs)


def _sel_mats(h):
    h1 = h // 2
    mats = []
    for d in range(4):
        s = np.zeros((h1, h), np.float32)
        for i in range(h1):
            j = 2 * i + d - 1
            if 0 <= j < h:
                s[i, j] = 1.0
        mats.append(s)
    return mats


def _comb_vecs(h1):
    h2 = h1 // 2
    vecs = []
    for d in range(4):
        u = np.zeros((1, h1), np.float32)
        for o in range(h2):
            j = 2 * o + d - 1
            if 0 <= j < h1:
                u[0, j] = 1.0
        vecs.append(u)
    return vecs


def _disc_logit(cmap, w1, b1, w2, b2, s_ref, u_ref, h):
    h1 = h // 2
    h2 = h // 4
    S = [s_ref[d * h1 : (d + 1) * h1, :] for d in range(4)]
    U = [u_ref[d : d + 1, :] for d in range(4)]

    y1 = jnp.zeros((_OC, h1, h1), jnp.float32)
    for dy in range(4):
        a = jnp.dot(S[dy], cmap)
        for dx in range(4):
            p = jnp.dot(a, S[dx].T)
            t = 4 * dy + dx
            y1 = y1 + p[None] * w1[:, t : t + 1][:, :, None]
    y1 = y1 + b1[:, :, None]
    y1 = jnp.where(y1 >= 0, y1, 0.2 * y1)

    acc = jnp.float32(0.0)
    for dy in range(4):
        r = jnp.sum(y1 * U[dy][:, :, None], axis=1)
        for dx in range(4):
            tv = jnp.sum(r * U[dx], axis=1, keepdims=True)
            acc = acc + jnp.sum(tv * w2[:, 4 * dy + dx : 4 * dy + dx + 1])
    return acc / jnp.float32(h2 * h2) + b2[0, 0]


def _bce_term(logit, tgt):
    return (
        jnp.maximum(logit, 0.0)
        - logit * tgt
        + jnp.log1p(jnp.exp(-jnp.abs(logit)))
    ) / jnp.float32(_B)


def _disc_body(xl_ref, xm_ref, xs_ref,
               w1l_ref, b1l_ref, w2l_ref, b2l_ref,
               w1m_ref, b1m_ref, w2m_ref, b2m_ref,
               w1s_ref, b1s_ref, w2s_ref, b2s_ref,
               lab_ref,
               sl_ref, ul_ref, sm_ref, um_ref, ss_ref, us_ref,
               ol_ref, om_ref, os_ref):
    i = pl.program_id(0)
    tgt = lab_ref[0, 0, 0]
    triples = (
        (xl_ref, w1l_ref, b1l_ref, w2l_ref, b2l_ref, sl_ref, ul_ref, ol_ref),
        (xm_ref, w1m_ref, b1m_ref, w2m_ref, b2m_ref, sm_ref, um_ref, om_ref),
        (xs_ref, w1s_ref, b1s_ref, w2s_ref, b2s_ref, ss_ref, us_ref, os_ref),
    )
    for h, (x_ref, w1_ref, b1_ref, w2_ref, b2_ref, s_ref, u_ref, o_ref) in zip(
        _HS, triples
    ):
        logit = _disc_logit(
            x_ref[0], w1_ref[...], b1_ref[...], w2_ref[...], b2_ref[...],
            s_ref, u_ref, h,
        )
        contrib2d = jnp.broadcast_to(_bce_term(logit, tgt), (1, 1))

        @pl.when(i == 0)
        def _():
            o_ref[...] = contrib2d

        @pl.when(i > 0)
        def _():
            o_ref[...] = o_ref[...] + contrib2d


def _disc_losses(maps, weights, labels):
    b = maps[0].shape[0]
    sel = []
    for h in _HS:
        sel.append(jnp.asarray(np.concatenate(_sel_mats(h), axis=0)))
        sel.append(jnp.asarray(np.concatenate(_comb_vecs(h // 2), axis=0)))
    in_specs = [
        pl.BlockSpec((1, h, h), lambda i: (i, 0, 0)) for h in _HS
    ]
    wargs = []
    for w1, b1, w2, b2 in weights:
        wargs += [w1, b1, w2, b2]
        in_specs += [
            pl.BlockSpec((_OC, 16), lambda i: (0, 0)),
            pl.BlockSpec((_OC, 1), lambda i: (0, 0)),
            pl.BlockSpec((_OC, 16), lambda i: (0, 0)),
            pl.BlockSpec((1, 1), lambda i: (0, 0)),
        ]
    in_specs.append(pl.BlockSpec((1, 1, 1), lambda i: (i, 0, 0)))
    for h in _HS:
        in_specs += [
            pl.BlockSpec((2 * h, h), lambda i: (0, 0)),
            pl.BlockSpec((4, h // 2), lambda i: (0, 0)),
        ]
    outs = pl.pallas_call(
        _disc_body,
        grid=(b,),
        in_specs=in_specs,
        out_specs=[pl.BlockSpec((1, 1), lambda i: (0, 0))] * 3,
        out_shape=[jax.ShapeDtypeStruct((1, 1), jnp.float32)] * 3,
    )(*maps, *wargs, labels, *sel)
    return tuple(o[0, 0] for o in outs)


def kernel(lf, mf, sf, da_preds, da_images, da_labels, device,
           lf_W1, lf_b1, lf_W2, lf_b2,
           mf_W1, mf_b1, mf_W2, mf_b2,
           sf_W1, sf_b1, sf_W2, sf_b2):
    labels = da_labels.astype(jnp.float32).reshape(_B, 1, 1)
    xs = [f.reshape(f.shape[0], f.shape[1], -1) for f in (lf, mf, sf)]
    centers = _kmeans_centers(*xs)
    maps = [
        c.reshape(f.shape[0], f.shape[2], f.shape[3])
        for c, f in zip(centers, (lf, mf, sf))
    ]
    weights = [
        (lf_W1.reshape(_OC, 16), lf_b1.reshape(_OC, 1),
         lf_W2.reshape(_OC, 16), lf_b2.reshape(1, 1)),
        (mf_W1.reshape(_OC, 16), mf_b1.reshape(_OC, 1),
         mf_W2.reshape(_OC, 16), mf_b2.reshape(1, 1)),
        (sf_W1.reshape(_OC, 16), sf_b1.reshape(_OC, 1),
         sf_W2.reshape(_OC, 16), sf_b2.reshape(1, 1)),
    ]
    return _disc_losses(maps, weights, labels)
